# baseline (speedup 1.0000x reference)
import functools

import jax
import jax.numpy as jnp
from jax import lax
from jax.experimental import pallas as pl
from jax.experimental.pallas import tpu as pltpu
from jax.experimental.pallas import tpu_sc as plsc

VOCAB = 1000000
EMBED_DIM = 32
SEQS = 16384
SEQ_LEN = 200
NC, NS, L = 2, 16, 16
NW = NC * NS
SEQS_PER_W = SEQS // NW
SB = 128
SB_PER_W = SEQS_PER_W // SB
PB = 5
PCHUNKS = SEQ_LEN // PB
CHUNKS_PER_W = SB_PER_W * PCHUNKS
CTILES = EMBED_DIM // 8
NBUF = 2


def _embed_kernel(ids_hbm, table_hbm, out_hbm, idx_sb, idx_cols, rows_v,
                  tiles_v, gsem, osem):
    wid = lax.axis_index("s") * NC + lax.axis_index("c")
    seq_base = wid * SEQS_PER_W
    st_base = wid * SB_PER_W

    zeros16 = jnp.zeros((L,), jnp.float32)
    svecs = [lax.iota(jnp.int32, L) + sg * L for sg in range(SB // L)]
    offs = [c * L for c in range(SEQ_LEN // L)] + [SEQ_LEN - L]

    def stage_idx_sb(stl):
        pltpu.sync_copy(ids_hbm.at[pl.ds(seq_base + stl * SB, SB)], idx_sb)

    def scan_sb():
        def row_scan(r, acc):
            for o in offs:
                acc = jnp.minimum(acc, idx_sb[r, pl.ds(o, L)])
            return acc
        acc = lax.fori_loop(0, SB, row_scan,
                            jnp.full((L,), VOCAB, jnp.int32))
        cnt = plsc.all_reduce_population_count(acc == 0)
        return (cnt[0] > 0).astype(jnp.int32)

    def build_chunk_idx(b, pq):
        bv = jnp.full((L,), b, jnp.int32)
        for pp in range(PB):
            pv = jnp.full((L,), pq * PB + pp, jnp.int32)
            for sg in range(SB // L):
                v = plsc.load_gather(idx_sb, [svecs[sg], pv])
                idx_cols[b, pp, pl.ds(sg * L, L)] = v

    def gather_chunk(b, fire):
        mk = pltpu.async_copy if fire else pltpu.make_async_copy
        for pp in range(PB):
            c = mk(table_hbm.at[idx_cols.at[b, pp]], rows_v.at[b, pp],
                   gsem)
            if not fire:
                c.wait()

    def fix_chunk(b):
        bv = jnp.full((L,), b, jnp.int32)

        def fix_p(pp, _):
            ppv = jnp.full((L,), pp, jnp.int32)
            for sg in range(SB // L):
                mask = idx_cols[b, pp, pl.ds(sg * L, L)] == 0
                for col in range(EMBED_DIM):
                    cv = jnp.full((L,), col, jnp.int32)
                    plsc.store_scatter(rows_v, [bv, ppv, svecs[sg], cv],
                                       zeros16, mask=mask)
            return _
        lax.fori_loop(0, PB, fix_p, 0)

    def transpose_chunk(b):
        bv = jnp.full((L,), b, jnp.int32)

        def tr_p(pp, _):
            ppv = jnp.full((L,), pp, jnp.int32)

            def tr_tc(tc, __):
                for c8 in range(8):
                    cv = jnp.full((L,), 0, jnp.int32) + (tc * 8 + c8)
                    for sg in range(SB // L):
                        v = plsc.load_gather(
                            rows_v, [bv, ppv, svecs[sg], cv])
                        tiles_v[b, pp, tc, c8, pl.ds(sg * L, L)] = v
                return __
            lax.fori_loop(0, CTILES, tr_tc, 0)
            return _
        lax.fori_loop(0, PB, tr_p, 0)

    def out_copies(g, b, fire):
        stl, pq = g // PCHUNKS, g % PCHUNKS
        mk = pltpu.async_copy if fire else pltpu.make_async_copy
        for pp in range(PB):
            for tc in range(CTILES):
                c = mk(tiles_v.at[b, pp, tc],
                       out_hbm.at[pq * PB + pp, tc, st_base + stl],
                       osem)
                if not fire:
                    c.wait()

    stage_idx_sb(0)
    hc0 = scan_sb()
    build_chunk_idx(0, 0)
    gather_chunk(0, fire=True)

    def chunk_body(g, hc):
        b = g % NBUF
        nb = (g + 1) % NBUF

        gather_chunk(b, fire=False)

        @pl.when(hc > 0)
        def _():
            fix_chunk(b)

        transpose_chunk(b)
        out_copies(g, b, fire=True)

        pq1 = (g + 1) % PCHUNKS

        @pl.when(g + 1 < CHUNKS_PER_W)
        def _n():
            @pl.when(g >= 1)
            def _w():
                out_copies(g - 1, nb, fire=False)

            @pl.when(pq1 == 0)
            def _s():
                stage_idx_sb((g + 1) // PCHUNKS)
            build_chunk_idx(nb, pq1)
            gather_chunk(nb, fire=True)

        hc_new = lax.cond(
            jnp.logical_and(pq1 == 0, g + 1 < CHUNKS_PER_W),
            scan_sb, lambda: hc)
        return hc_new

    lax.fori_loop(0, CHUNKS_PER_W, chunk_body, hc0)

    gl = CHUNKS_PER_W - 1
    out_copies(gl - 1, (gl - 1) % NBUF, fire=False)
    out_copies(gl, gl % NBUF, fire=False)


@functools.partial(jax.jit, static_argnames=())
def kernel(input_ids, table):
    mesh = plsc.VectorSubcoreMesh(core_axis_name="c", subcore_axis_name="s",
                                  num_cores=NC, num_subcores=NS)
    out5 = pl.kernel(
        _embed_kernel,
        out_type=jax.ShapeDtypeStruct(
            (SEQ_LEN, CTILES, SEQS // SB, 8, SB), jnp.float32),
        mesh=mesh,
        scratch_types=[
            pltpu.VMEM((SB, SEQ_LEN), jnp.int32),
            pltpu.VMEM((NBUF, PB, SB), jnp.int32),
            pltpu.VMEM((NBUF, PB, SB, EMBED_DIM), jnp.float32),
            pltpu.VMEM((NBUF, PB, CTILES, 8, SB), jnp.float32),
            pltpu.SemaphoreType.DMA,
            pltpu.SemaphoreType.DMA,
        ],
        compiler_params=pltpu.CompilerParams(needs_layout_passes=False,
                                             use_tc_tiling_on_sc=False),
    )(input_ids, table)
    out = jnp.transpose(out5, (2, 4, 0, 1, 3))
    return out.reshape(SEQS, SEQ_LEN, EMBED_DIM)


# speedup vs baseline: 1.0089x; 1.0089x over previous
import functools

import jax
import jax.numpy as jnp
from jax import lax
from jax.experimental import pallas as pl
from jax.experimental.pallas import tpu as pltpu
from jax.experimental.pallas import tpu_sc as plsc

VOCAB = 1000000
EMBED_DIM = 32
SEQS = 16384
SEQ_LEN = 200
NC, NS, L = 2, 16, 16
NW = NC * NS
SEQS_PER_W = SEQS // NW
SB = 128
SB_PER_W = SEQS_PER_W // SB
PB = 5
PCHUNKS = SEQ_LEN // PB
CHUNKS_PER_W = SB_PER_W * PCHUNKS
CTILES = EMBED_DIM // 8
NBUF = 2


def _embed_kernel(ids_hbm, table_hbm, out_hbm, idx_sb, idx_cols, rows_v,
                  tiles_v, gsem, osem):
    wid = lax.axis_index("s") * NC + lax.axis_index("c")
    seq_base = wid * SEQS_PER_W
    st_base = wid * SB_PER_W

    zeros16 = jnp.zeros((L,), jnp.float32)
    svecs = [lax.iota(jnp.int32, L) + sg * L for sg in range(SB // L)]
    cvecs = [jnp.full((L,), c, jnp.int32) for c in range(EMBED_DIM)]
    offs = [c * L for c in range(SEQ_LEN // L)] + [SEQ_LEN - L]

    def stage_idx_sb(stl):
        pltpu.sync_copy(ids_hbm.at[pl.ds(seq_base + stl * SB, SB)], idx_sb)

    def scan_sb():
        def row_scan(r, acc):
            for o in offs:
                acc = jnp.minimum(acc, idx_sb[r, pl.ds(o, L)])
            return acc
        acc = lax.fori_loop(0, SB, row_scan,
                            jnp.full((L,), VOCAB, jnp.int32))
        cnt = plsc.all_reduce_population_count(acc == 0)
        return (cnt[0] > 0).astype(jnp.int32)

    def build_chunk_idx(b, pq):
        for pp in range(PB):
            pv = jnp.full((L,), pq * PB + pp, jnp.int32)
            for sg in range(SB // L):
                v = plsc.load_gather(idx_sb, [svecs[sg], pv])
                idx_cols[b, pp, pl.ds(sg * L, L)] = v

    def gather_chunk(b, fire):
        mk = pltpu.async_copy if fire else pltpu.make_async_copy
        for pp in range(PB):
            c = mk(table_hbm.at[idx_cols.at[b, pp]],
                   rows_v.at[pl.ds((b * PB + pp) * SB, SB)],
                   gsem)
            if not fire:
                c.wait()

    def fix_chunk(b):
        def fix_p(pp, _):
            rb = jnp.full((L,), (b * PB + pp) * SB, jnp.int32)
            for sg in range(SB // L):
                mask = idx_cols[b, pp, pl.ds(sg * L, L)] == 0
                rv = rb + svecs[sg]
                for col in range(EMBED_DIM):
                    plsc.store_scatter(rows_v, [rv, cvecs[col]],
                                       zeros16, mask=mask)
            return _
        lax.fori_loop(0, PB, fix_p, 0)

    def transpose_chunk(b):
        def tr_p(pp, _):
            rb = jnp.full((L,), (b * PB + pp) * SB, jnp.int32)
            rvs = [rb + svecs[sg] for sg in range(SB // L)]
            for tc in range(CTILES):
                for c8 in range(8):
                    cv = cvecs[tc * 8 + c8]
                    for sg in range(SB // L):
                        v = plsc.load_gather(rows_v, [rvs[sg], cv])
                        tiles_v[b, pp, tc, c8, pl.ds(sg * L, L)] = v
            return _
        lax.fori_loop(0, PB, tr_p, 0)

    def out_copies(g, b, fire):
        stl, pq = g // PCHUNKS, g % PCHUNKS
        mk = pltpu.async_copy if fire else pltpu.make_async_copy
        for pp in range(PB):
            c = mk(tiles_v.at[b, pp],
                   out_hbm.at[pq * PB + pp, pl.ds(0, CTILES),
                              st_base + stl],
                   osem)
            if not fire:
                c.wait()

    stage_idx_sb(0)
    hc0 = scan_sb()
    build_chunk_idx(0, 0)
    gather_chunk(0, fire=True)

    def chunk_body(g, hc):
        b = g % NBUF
        nb = (g + 1) % NBUF

        gather_chunk(b, fire=False)

        @pl.when(hc > 0)
        def _():
            fix_chunk(b)

        transpose_chunk(b)
        out_copies(g, b, fire=True)

        pq1 = (g + 1) % PCHUNKS

        @pl.when(g + 1 < CHUNKS_PER_W)
        def _n():
            @pl.when(g >= 1)
            def _w():
                out_copies(g - 1, nb, fire=False)

            @pl.when(pq1 == 0)
            def _s():
                stage_idx_sb((g + 1) // PCHUNKS)
            build_chunk_idx(nb, pq1)
            gather_chunk(nb, fire=True)

        hc_new = lax.cond(
            jnp.logical_and(pq1 == 0, g + 1 < CHUNKS_PER_W),
            scan_sb, lambda: hc)
        return hc_new

    lax.fori_loop(0, CHUNKS_PER_W, chunk_body, hc0)

    gl = CHUNKS_PER_W - 1
    out_copies(gl - 1, (gl - 1) % NBUF, fire=False)
    out_copies(gl, gl % NBUF, fire=False)


@functools.partial(jax.jit, static_argnames=())
def kernel(input_ids, table):
    mesh = plsc.VectorSubcoreMesh(core_axis_name="c", subcore_axis_name="s",
                                  num_cores=NC, num_subcores=NS)
    out5 = pl.kernel(
        _embed_kernel,
        out_type=jax.ShapeDtypeStruct(
            (SEQ_LEN, CTILES, SEQS // SB, 8, SB), jnp.float32),
        mesh=mesh,
        scratch_types=[
            pltpu.VMEM((SB, SEQ_LEN), jnp.int32),
            pltpu.VMEM((NBUF, PB, SB), jnp.int32),
            pltpu.VMEM((NBUF * PB * SB, EMBED_DIM), jnp.float32),
            pltpu.VMEM((NBUF, PB, CTILES, 8, SB), jnp.float32),
            pltpu.SemaphoreType.DMA,
            pltpu.SemaphoreType.DMA,
        ],
        compiler_params=pltpu.CompilerParams(needs_layout_passes=False,
                                             use_tc_tiling_on_sc=False),
    )(input_ids, table)
    out = jnp.transpose(out5, (2, 4, 0, 1, 3))
    return out.reshape(SEQS, SEQ_LEN, EMBED_DIM)


# speedup vs baseline: 1.3723x; 1.3601x over previous
import functools

import jax
import jax.numpy as jnp
from jax import lax
from jax.experimental import pallas as pl
from jax.experimental.pallas import tpu as pltpu
from jax.experimental.pallas import tpu_sc as plsc

VOCAB = 1000000
EMBED_DIM = 32
SEQS = 16384
SEQ_LEN = 200
NC, NS, L = 2, 16, 16
NW = NC * NS
SEQS_PER_W = SEQS // NW
NSEQ = 8
CHUNKS_PER_W = SEQS_PER_W // NSEQ
GSPLIT = (128, 72)
NBUF = 2


def _embed_kernel(ids_hbm, table_hbm, out_hbm, idx_v, rows_v, gsem, osem):
    wid = lax.axis_index("s") * NC + lax.axis_index("c")
    seq_base = wid * SEQS_PER_W

    zeros16 = jnp.zeros((L,), jnp.float32)
    lane = lax.iota(jnp.int32, L)
    offs = [c * L for c in range(SEQ_LEN // L)] + [SEQ_LEN - L]

    def stage_idx(g, b):
        pltpu.sync_copy(ids_hbm.at[pl.ds(seq_base + g * NSEQ, NSEQ)],
                        idx_v.at[b])

    def gather_copies(b, make_only):
        mk = pltpu.make_async_copy if make_only else pltpu.async_copy
        for j in range(NSEQ):
            o = 0
            for glen in GSPLIT:
                mk(table_hbm.at[idx_v.at[b, j, pl.ds(o, glen)]],
                   rows_v.at[b, j, pl.ds(o, glen)],
                   gsem)
                o += glen

    def fire_gathers(b):
        gather_copies(b, make_only=False)

    def drain_gathers(b):
        for j in range(NSEQ):
            o = 0
            for glen in GSPLIT:
                pltpu.make_async_copy(
                    table_hbm.at[idx_v.at[b, j, pl.ds(o, glen)]],
                    rows_v.at[b, j, pl.ds(o, glen)],
                    gsem).wait()
                o += glen

    def out_slice(g):
        return out_hbm.at[pl.ds(seq_base + g * NSEQ, NSEQ)]

    def fire_out(g, b):
        pltpu.async_copy(rows_v.at[b], out_slice(g), osem)

    def wait_out(g, b):
        pltpu.make_async_copy(rows_v.at[b], out_slice(g), osem).wait()

    def scan_and_fix(b):
        acc = jnp.full((L,), VOCAB, jnp.int32)
        for j in range(NSEQ):
            for o in offs:
                acc = jnp.minimum(acc, idx_v[b, j, pl.ds(o, L)])
        cnt = plsc.all_reduce_population_count(acc == 0)
        has_pad = cnt[0] > 0

        @pl.when(has_pad)
        def _fix():
            bvec = jnp.full((L,), b, jnp.int32)

            def fix_row(j, _):
                jvec = jnp.full((L,), j, jnp.int32)
                for o in offs:
                    vec = idx_v[b, j, pl.ds(o, L)]
                    mask = vec == 0
                    row_ids = o + lane
                    for col in range(EMBED_DIM):
                        col_ids = jnp.full((L,), col, jnp.int32)
                        plsc.store_scatter(rows_v,
                                           [bvec, jvec, row_ids, col_ids],
                                           zeros16, mask=mask)
                return _
            lax.fori_loop(0, NSEQ, fix_row, 0)

    stage_idx(0, 0)
    fire_gathers(0)

    def chunk_body(g, _):
        b = g % NBUF
        nb = (g + 1) % NBUF

        drain_gathers(b)
        scan_and_fix(b)
        fire_out(g, b)

        @pl.when(g + 1 < CHUNKS_PER_W)
        def _next():
            @pl.when(g >= 1)
            def _w():
                wait_out(g - 1, nb)
            stage_idx(g + 1, nb)
            fire_gathers(nb)
        return _

    lax.fori_loop(0, CHUNKS_PER_W, chunk_body, 0)

    gl = CHUNKS_PER_W - 1
    wait_out(gl - 1, (gl - 1) % NBUF)
    wait_out(gl, gl % NBUF)


@functools.partial(jax.jit, static_argnames=())
def kernel(input_ids, table):
    mesh = plsc.VectorSubcoreMesh(core_axis_name="c", subcore_axis_name="s",
                                  num_cores=NC, num_subcores=NS)
    out = pl.kernel(
        _embed_kernel,
        out_type=jax.ShapeDtypeStruct((SEQS, SEQ_LEN, EMBED_DIM),
                                      jnp.float32),
        mesh=mesh,
        scratch_types=[
            pltpu.VMEM((NBUF, NSEQ, SEQ_LEN), jnp.int32),
            pltpu.VMEM((NBUF, NSEQ, SEQ_LEN, EMBED_DIM), jnp.float32),
            pltpu.SemaphoreType.DMA,
            pltpu.SemaphoreType.DMA,
        ],
        compiler_params=pltpu.CompilerParams(needs_layout_passes=False,
                                             use_tc_tiling_on_sc=False),
    )(input_ids, table)
    return out
